# Initial kernel scaffold; baseline (speedup 1.0000x reference)
import jax
import jax.numpy as jnp
from jax.experimental import pallas as pl


def kernel(x, edge_index, batch, W1, a1s, a1d, b1, W2, a2s, a2d, b2, Wl, bl):
    raise NotImplementedError("write your pallas kernel here")


# speedup vs baseline: 1.0045x; 1.0045x over previous
import jax
import jax.numpy as jnp
from jax.experimental import pallas as pl

_N = 50000
_E = 800000
_G = 256
_H = 64
_BLK = 2000


def _mm_body(x_ref, w_ref, o_ref):
    o_ref[...] = jnp.dot(x_ref[...], w_ref[...],
                         preferred_element_type=jnp.float32)


def _linear_aug(x, waug):
    n, d = x.shape
    return pl.pallas_call(
        _mm_body,
        grid=(n // _BLK,),
        in_specs=[
            pl.BlockSpec((_BLK, d), lambda i: (i, 0)),
            pl.BlockSpec((d, 128), lambda i: (0, 0)),
        ],
        out_specs=pl.BlockSpec((_BLK, 128), lambda i: (i, 0)),
        out_shape=jax.ShapeDtypeStruct((n, 128), jnp.float32),
    )(x, waug)


def _gat_layer(feats, W, a_s, a_d, b, src, dst):
    d = feats.shape[1]
    c_s = a_s @ W
    c_d = a_d @ W
    waug = (jnp.zeros((d, 128), jnp.float32)
            .at[:, :_H].set(W.T).at[:, _H].set(c_s).at[:, _H + 1].set(c_d))
    out = _linear_aug(feats, waug)
    h = out[:, :_H]
    asrc = out[:, _H]
    adst = out[:, _H + 1]
    e = jax.nn.leaky_relu(asrc[src] + adst[dst], negative_slope=0.2)
    e_max = jax.ops.segment_max(e, dst, num_segments=_N)
    e_exp = jnp.exp(e - e_max[dst])
    denom = jax.ops.segment_sum(e_exp, dst, num_segments=_N)
    alpha = e_exp / (denom[dst] + 1e-16)
    msg = h[src] * alpha[:, None]
    agg = jax.ops.segment_sum(msg, dst, num_segments=_N)
    return agg + b


def kernel(x, edge_index, batch, W1, a1s, a1d, b1, W2, a2s, a2d, b2, Wl, bl):
    src = edge_index[0]
    dst = edge_index[1]
    h = _gat_layer(x, W1, a1s, a1d, b1, src, dst)
    h = jax.nn.relu(h)
    h = _gat_layer(h, W2, a2s, a2d, b2, src, dst)
    h = jax.nn.relu(h)
    s = jax.ops.segment_sum(h, batch, num_segments=_G)
    cnt = jax.ops.segment_sum(jnp.ones((_N,), jnp.float32), batch,
                              num_segments=_G)
    g = s / jnp.maximum(cnt, 1.0)[:, None]
    return g @ Wl.T + bl


# speedup vs baseline: 17.0440x; 16.9683x over previous
import functools

import jax
import jax.numpy as jnp
from jax import lax
from jax.experimental import pallas as pl
from jax.experimental.pallas import tpu as pltpu
from jax.experimental.pallas import tpu_sc as plsc

_N = 50000
_E = 800000
_G = 256
_H = 64
_BLK = 2000

_NC, _NS, _L = 2, 16, 16
_NP = 50048
_TRASH = _N
_EPT = 51200
_EP = _EPT * _NS
_CHUNK = 128
_NCHUNK = _EPT // _CHUNK
_NHALF = _NP // 2
_NDP = _NHALF + 64


def _mm_body(x_ref, w_ref, o_ref):
    o_ref[...] = jnp.dot(x_ref[...], w_ref[...],
                         preferred_element_type=jnp.float32)


def _linear_aug(x, waug):
    n, d = x.shape
    return pl.pallas_call(
        _mm_body,
        grid=(n // _BLK,),
        in_specs=[
            pl.BlockSpec((_BLK, d), lambda i: (i, 0)),
            pl.BlockSpec((d, 128), lambda i: (0, 0)),
        ],
        out_specs=pl.BlockSpec((_BLK, 128), lambda i: (i, 0)),
        out_shape=jax.ShapeDtypeStruct((n, 128), jnp.float32),
    )(x, waug)


def _sc_body(hv_h, asrc_h, adst_h, srcp_h, dstp_h, out_h, den_h,
             accum, den, asrc_s, adst_s, srci, dsti, dstloc, idx2, wbuf,
             ae, be, rows, sem):
    cid = lax.axis_index("c")
    sid = lax.axis_index("s")
    zero = jnp.zeros((_L,), jnp.float32)

    def _zb(r, _):
        rows[r, 0:16] = zero
        rows[r, 16:32] = zero
        return _
    lax.fori_loop(0, _CHUNK, _zb, None)
    for k in range(_CHUNK // _L):
        wbuf[pl.ds(k * _L, _L)] = zero

    arow = _NP // _NS
    for t in range(arow // _CHUNK):
        pltpu.sync_copy(rows, accum.at[pl.ds(sid * arow + t * _CHUNK,
                                             _CHUNK)])
    _atail = arow % _CHUNK
    if _atail:
        pltpu.sync_copy(rows.at[pl.ds(0, _atail)],
                        accum.at[pl.ds(sid * arow + arow - _atail, _atail)])

    drow = _NDP // _NS
    for t in range(drow // _CHUNK):
        pltpu.sync_copy(wbuf, den.at[pl.ds(sid * drow + t * _CHUNK, _CHUNK)])
    _dtail = drow % _CHUNK
    if _dtail:
        pltpu.sync_copy(wbuf.at[pl.ds(0, _dtail)],
                        den.at[pl.ds(sid * drow + drow - _dtail, _dtail)])

    @pl.when(sid == 0)
    def _stage():
        pltpu.sync_copy(asrc_h, asrc_s.at[pl.ds(0, _N)])
        pltpu.sync_copy(adst_h, adst_s.at[pl.ds(0, _N)])
        pltpu.sync_copy(wbuf.at[pl.ds(0, _NP - _N)],
                        asrc_s.at[pl.ds(_N, _NP - _N)])
        pltpu.sync_copy(wbuf.at[pl.ds(0, _NP - _N)],
                        adst_s.at[pl.ds(_N, _NP - _N)])

    plsc.subcore_barrier()

    ebase = sid * _EPT
    dbase = cid * _NHALF

    def _chunk(j, _):
        off = ebase + j * _CHUNK
        pltpu.sync_copy(srcp_h.at[pl.ds(off, _CHUNK)], srci)
        pltpu.sync_copy(dstp_h.at[pl.ds(off, _CHUNK)], dsti.at[0])
        pltpu.async_copy(asrc_s.at[srci], ae, sem).wait()
        pltpu.async_copy(adst_s.at[dsti.at[0]], be, sem).wait()
        for k in range(_CHUNK // _L):
            s16 = srci[pl.ds(k * _L, _L)]
            d16 = dsti[0, pl.ds(k * _L, _L)]
            x = ae[pl.ds(k * _L, _L)] + be[pl.ds(k * _L, _L)]
            w = jnp.exp(jnp.where(x >= 0, x, 0.2 * x))
            wbuf[pl.ds(k * _L, _L)] = w
            idx2[pl.ds(k * _L, _L)] = s16 * 2 + cid
            dl = d16 - dbase
            inr = (dl >= 0) & (dl < _NHALF)
            dstloc[0, pl.ds(k * _L, _L)] = jnp.where(inr, dl, _NHALF)
        pltpu.async_copy(hv_h.at[idx2], rows, sem).wait()
        for k in range(_CHUNK // _L):
            wv = wbuf[pl.ds(k * _L, _L)]
            for t in range(_L):
                r = k * _L + t
                ws = jnp.broadcast_to(wv[t], (_L,))
                rows[r, 0:16] = rows[r, 0:16] * ws
                rows[r, 16:32] = rows[r, 16:32] * ws
        pltpu.sync_copy(rows, accum.at[dsti.at[0]], add=True)
        pltpu.sync_copy(wbuf, den.at[dstloc.at[0]], add=True)
        return _

    lax.fori_loop(0, _NCHUNK, _chunk, None)

    plsc.subcore_barrier()

    pltpu.sync_copy(accum.at[pl.ds(sid * arow, arow)],
                    out_h.at[cid, pl.ds(sid * arow, arow)])
    pltpu.sync_copy(den.at[pl.ds(sid * drow, drow)],
                    den_h.at[cid, pl.ds(sid * drow, drow)])


def _sc_edge(hv, asrc, adst, srcp, dstp):
    return pl.kernel(
        _sc_body,
        out_type=[jax.ShapeDtypeStruct((_NC, _NP, 32), jnp.float32),
                  jax.ShapeDtypeStruct((_NC, _NDP), jnp.float32)],
        mesh=plsc.VectorSubcoreMesh(core_axis_name="c", subcore_axis_name="s"),
        compiler_params=pltpu.CompilerParams(needs_layout_passes=False,
                                             use_tc_tiling_on_sc=False),
        scratch_types=[
            pltpu.VMEM_SHARED((_NP, 32), jnp.float32),
            pltpu.VMEM_SHARED((_NDP,), jnp.float32),
            pltpu.VMEM_SHARED((_NP,), jnp.float32),
            pltpu.VMEM_SHARED((_NP,), jnp.float32),
            pltpu.VMEM((_CHUNK,), jnp.int32),
            pltpu.VMEM((1, _CHUNK), jnp.int32),
            pltpu.VMEM((1, _CHUNK), jnp.int32),
            pltpu.VMEM((_CHUNK,), jnp.int32),
            pltpu.VMEM((_CHUNK,), jnp.float32),
            pltpu.VMEM((_CHUNK,), jnp.float32),
            pltpu.VMEM((_CHUNK,), jnp.float32),
            pltpu.VMEM((_CHUNK, 32), jnp.float32),
            pltpu.SemaphoreType.DMA,
        ],
    )(hv, asrc, adst, srcp, dstp)


def _gat_layer(feats, W, a_s, a_d, b, srcp, dstp):
    d = feats.shape[1]
    c_s = a_s @ W
    c_d = a_d @ W
    waug = (jnp.zeros((d, 128), jnp.float32)
            .at[:, :_H].set(W.T).at[:, _H].set(c_s).at[:, _H + 1].set(c_d))
    out = _linear_aug(feats, waug)
    h = out[:, :_H]
    asrc = out[:, _H]
    adst = out[:, _H + 1]
    hv = h.reshape(2 * _N, 32)
    agg2, den2 = _sc_edge(hv, asrc, adst, srcp, dstp)
    agg = jnp.concatenate([agg2[0, :_N], agg2[1, :_N]], axis=1)
    den = jnp.concatenate([den2[0, :_NHALF], den2[1, :_NHALF]])[:_N]
    return agg / (den + 1e-16)[:, None] + b


def kernel(x, edge_index, batch, W1, a1s, a1d, b1, W2, a2s, a2d, b2, Wl, bl):
    src = edge_index[0]
    dst = edge_index[1]
    srcp = jnp.pad(src, (0, _EP - _E))
    dstp = jnp.pad(dst, (0, _EP - _E), constant_values=_TRASH)
    h = _gat_layer(x, W1, a1s, a1d, b1, srcp, dstp)
    h = jax.nn.relu(h)
    h = _gat_layer(h, W2, a2s, a2d, b2, srcp, dstp)
    h = jax.nn.relu(h)
    s = jax.ops.segment_sum(h, batch, num_segments=_G, indices_are_sorted=True)
    cnt = jax.ops.segment_sum(jnp.ones((_N,), jnp.float32), batch,
                              num_segments=_G, indices_are_sorted=True)
    g = s / jnp.maximum(cnt, 1.0)[:, None]
    return g @ Wl.T + bl


# speedup vs baseline: 24.7704x; 1.4533x over previous
import functools

import jax
import jax.numpy as jnp
from jax import lax
from jax.experimental import pallas as pl
from jax.experimental.pallas import tpu as pltpu
from jax.experimental.pallas import tpu_sc as plsc

_N = 50000
_E = 800000
_G = 256
_H = 64
_BLK = 2000

_NC, _NS, _L = 2, 16, 16
_NP = 50048
_TRASH = _N
_EPT = 51200
_EP = _EPT * _NS
_CHUNK = 128
_NCHUNK = _EPT // _CHUNK
_NHALF = _NP // 2
_NDP = _NHALF + 64


def _mm_body(x_ref, w_ref, o_ref):
    o_ref[...] = jnp.dot(x_ref[...], w_ref[...],
                         preferred_element_type=jnp.float32)


def _linear_aug(x, waug):
    n, d = x.shape
    return pl.pallas_call(
        _mm_body,
        grid=(n // _BLK,),
        in_specs=[
            pl.BlockSpec((_BLK, d), lambda i: (i, 0)),
            pl.BlockSpec((d, 128), lambda i: (0, 0)),
        ],
        out_specs=pl.BlockSpec((_BLK, 128), lambda i: (i, 0)),
        out_shape=jax.ShapeDtypeStruct((n, 128), jnp.float32),
    )(x, waug)


def _sc_body(hv_h, asrc_h, adst_h, srcp_h, dstp2_h, out_h, den_h,
             accum, den, asrc_s, adst_s, srcw, dstw, dstloc, idx2, wbuf,
             ae, be, rows, sem_ab0, sem_ab1, sem_r0, sem_r1):
    cid = lax.axis_index("c")
    sid = lax.axis_index("s")
    zero = jnp.zeros((_L,), jnp.float32)

    def _zb(r, _):
        rows[0, r, 0:16] = zero
        rows[0, r, 16:32] = zero
        return _
    lax.fori_loop(0, _CHUNK, _zb, None)
    for k in range(_CHUNK // _L):
        wbuf[0, pl.ds(k * _L, _L)] = zero

    arow = _NP // _NS
    for t in range(arow // _CHUNK):
        pltpu.sync_copy(rows.at[0], accum.at[pl.ds(sid * arow + t * _CHUNK,
                                                   _CHUNK)])
    _atail = arow % _CHUNK
    if _atail:
        pltpu.sync_copy(rows.at[0, pl.ds(0, _atail)],
                        accum.at[pl.ds(sid * arow + arow - _atail, _atail)])

    drow = _NDP // _NS
    for t in range(drow // _CHUNK):
        pltpu.sync_copy(wbuf.at[0],
                        den.at[pl.ds(sid * drow + t * _CHUNK, _CHUNK)])
    _dtail = drow % _CHUNK
    if _dtail:
        pltpu.sync_copy(wbuf.at[0, pl.ds(0, _dtail)],
                        den.at[pl.ds(sid * drow + drow - _dtail, _dtail)])

    @pl.when(sid == 0)
    def _stage():
        pltpu.sync_copy(asrc_h, asrc_s.at[pl.ds(0, _N)])
        pltpu.sync_copy(adst_h, adst_s.at[pl.ds(0, _N)])
        pltpu.sync_copy(wbuf.at[0, pl.ds(0, _NP - _N)],
                        asrc_s.at[pl.ds(_N, _NP - _N)])
        pltpu.sync_copy(wbuf.at[0, pl.ds(0, _NP - _N)],
                        adst_s.at[pl.ds(_N, _NP - _N)])

    plsc.subcore_barrier()

    ebase = sid * _EPT
    dbase = cid * _NHALF
    _SUP = 8
    _SE = _SUP * _CHUNK

    def _compute(jj, p):
        for k in range(_CHUNK // _L):
            s16 = srcw[pl.ds(jj * _CHUNK + k * _L, _L)]
            d16 = dstw[jj, pl.ds(k * _L, _L)]
            x = ae[p, pl.ds(k * _L, _L)] + be[p, pl.ds(k * _L, _L)]
            w = jnp.exp(jnp.where(x >= 0, x, 0.2 * x))
            wbuf[p, pl.ds(k * _L, _L)] = w
            idx2[p, pl.ds(k * _L, _L)] = s16 * 2 + cid
            dl = d16 - dbase
            inr = (dl >= 0) & (dl < _NHALF)
            dstloc[p, pl.ds(k * _L, _L)] = jnp.where(inr, dl, _NHALF)

    def _scale(q):
        def _srow(g, _):
            wv = wbuf[q, pl.ds(g * _L, _L)]
            for t in range(_L):
                r = g * _L + t
                ws = jnp.broadcast_to(wv[t], (_L,))
                rows[q, r, 0:16] = rows[q, r, 0:16] * ws
                rows[q, r, 16:32] = rows[q, r, 16:32] * ws
            return _
        lax.fori_loop(0, _CHUNK // _L, _srow, None)

    def _issue_ab(jj):
        p = jj % 2
        da = pltpu.async_copy(
            asrc_s.at[srcw.at[pl.ds(jj * _CHUNK, _CHUNK)]],
            ae.at[p], sem_ab0 if p == 0 else sem_ab1)
        db = pltpu.async_copy(
            adst_s.at[dstw.at[jj]],
            be.at[p], sem_ab0 if p == 0 else sem_ab1)
        return da, db

    def _sup(s, _):
        base = ebase + s * _SE
        pltpu.sync_copy(srcp_h.at[pl.ds(base, _SE)], srcw)
        pltpu.sync_copy(dstp2_h.at[pl.ds(sid * _NCHUNK + s * _SUP, _SUP)],
                        dstw)
        descs_ab = [None] * _SUP
        descs_r = [None, None]
        descs_ab[0] = _issue_ab(0)
        for jj in range(_SUP):
            p = jj % 2
            if jj + 1 < _SUP:
                descs_ab[jj + 1] = _issue_ab(jj + 1)
            descs_ab[jj][0].wait()
            descs_ab[jj][1].wait()
            _compute(jj, p)
            descs_r[p] = pltpu.async_copy(
                hv_h.at[idx2.at[p]], rows.at[p],
                sem_r0 if p == 0 else sem_r1)
            if jj > 0:
                q = (jj - 1) % 2
                descs_r[q].wait()
                _scale(q)
                pltpu.sync_copy(rows.at[q], accum.at[dstw.at[jj - 1]],
                                add=True)
                pltpu.sync_copy(wbuf.at[q], den.at[dstloc.at[q]], add=True)
        q = (_SUP - 1) % 2
        descs_r[q].wait()
        _scale(q)
        pltpu.sync_copy(rows.at[q], accum.at[dstw.at[_SUP - 1]], add=True)
        pltpu.sync_copy(wbuf.at[q], den.at[dstloc.at[q]], add=True)
        return _

    lax.fori_loop(0, _EPT // _SE, _sup, None)

    plsc.subcore_barrier()

    pltpu.sync_copy(accum.at[pl.ds(sid * arow, arow)],
                    out_h.at[cid, pl.ds(sid * arow, arow)])
    pltpu.sync_copy(den.at[pl.ds(sid * drow, drow)],
                    den_h.at[cid, pl.ds(sid * drow, drow)])


def _sc_edge(hv, asrc, adst, srcp, dstp2):
    return pl.kernel(
        _sc_body,
        out_type=[jax.ShapeDtypeStruct((_NC, _NP, 32), jnp.float32),
                  jax.ShapeDtypeStruct((_NC, _NDP), jnp.float32)],
        mesh=plsc.VectorSubcoreMesh(core_axis_name="c", subcore_axis_name="s"),
        compiler_params=pltpu.CompilerParams(needs_layout_passes=False,
                                             use_tc_tiling_on_sc=False),
        scratch_types=[
            pltpu.VMEM_SHARED((_NP, 32), jnp.float32),
            pltpu.VMEM_SHARED((_NDP,), jnp.float32),
            pltpu.VMEM_SHARED((_NP,), jnp.float32),
            pltpu.VMEM_SHARED((_NP,), jnp.float32),
            pltpu.VMEM((8 * _CHUNK,), jnp.int32),
            pltpu.VMEM((8, _CHUNK), jnp.int32),
            pltpu.VMEM((2, _CHUNK), jnp.int32),
            pltpu.VMEM((2, _CHUNK), jnp.int32),
            pltpu.VMEM((2, _CHUNK), jnp.float32),
            pltpu.VMEM((2, _CHUNK), jnp.float32),
            pltpu.VMEM((2, _CHUNK), jnp.float32),
            pltpu.VMEM((2, _CHUNK, 32), jnp.float32),
            pltpu.SemaphoreType.DMA,
            pltpu.SemaphoreType.DMA,
            pltpu.SemaphoreType.DMA,
            pltpu.SemaphoreType.DMA,
        ],
    )(hv, asrc, adst, srcp, dstp2)


def _gat_layer(feats, W, a_s, a_d, b, srcp, dstp2):
    d = feats.shape[1]
    c_s = a_s @ W
    c_d = a_d @ W
    waug = (jnp.zeros((d, 128), jnp.float32)
            .at[:, :_H].set(W.T).at[:, _H].set(c_s).at[:, _H + 1].set(c_d))
    out = _linear_aug(feats, waug)
    h = out[:, :_H]
    asrc = out[:, _H]
    adst = out[:, _H + 1]
    hv = h.reshape(2 * _N, 32)
    agg2, den2 = _sc_edge(hv, asrc, adst, srcp, dstp2)
    agg = jnp.concatenate([agg2[0, :_N], agg2[1, :_N]], axis=1)
    den = jnp.concatenate([den2[0, :_NHALF], den2[1, :_NHALF]])[:_N]
    return agg / (den + 1e-16)[:, None] + b


def kernel(x, edge_index, batch, W1, a1s, a1d, b1, W2, a2s, a2d, b2, Wl, bl):
    src = edge_index[0]
    dst = edge_index[1]
    srcp = jnp.pad(src, (0, _EP - _E))
    dstp2 = jnp.pad(dst, (0, _EP - _E),
                    constant_values=_TRASH).reshape(_EP // _CHUNK, _CHUNK)
    h = _gat_layer(x, W1, a1s, a1d, b1, srcp, dstp2)
    h = jax.nn.relu(h)
    h = _gat_layer(h, W2, a2s, a2d, b2, srcp, dstp2)
    h = jax.nn.relu(h)
    s = jax.ops.segment_sum(h, batch, num_segments=_G, indices_are_sorted=True)
    cnt = jax.ops.segment_sum(jnp.ones((_N,), jnp.float32), batch,
                              num_segments=_G, indices_are_sorted=True)
    g = s / jnp.maximum(cnt, 1.0)[:, None]
    return g @ Wl.T + bl


# speedup vs baseline: 29.0688x; 1.1735x over previous
import functools

import jax
import jax.numpy as jnp
from jax import lax
from jax.experimental import pallas as pl
from jax.experimental.pallas import tpu as pltpu
from jax.experimental.pallas import tpu_sc as plsc

_N = 50000
_E = 800000
_G = 256
_H = 64
_BLK = 2000

_NC, _NS, _L = 2, 16, 16
_NP = 50048
_TRASH = _N
_EPT = 51200
_EP = _EPT * _NS
_CHUNK = 128
_NCHUNK = _EPT // _CHUNK
_NHALF = _NP // 2
_NDP = _NHALF + 64


def _mm_body(x_ref, w_ref, o_ref):
    o_ref[...] = jnp.dot(x_ref[...], w_ref[...],
                         preferred_element_type=jnp.float32)


def _linear_aug(x, waug):
    n, d = x.shape
    return pl.pallas_call(
        _mm_body,
        grid=(n // _BLK,),
        in_specs=[
            pl.BlockSpec((_BLK, d), lambda i: (i, 0)),
            pl.BlockSpec((d, 128), lambda i: (0, 0)),
        ],
        out_specs=pl.BlockSpec((_BLK, 128), lambda i: (i, 0)),
        out_shape=jax.ShapeDtypeStruct((n, 128), jnp.float32),
    )(x, waug)


def _sc_body(hv_h, asrc_h, adst_h, srcp_h, dstp2_h, out_h, den_h,
             accum, den, asrc_s, adst_s, srcw, dstw, dstloc, idx2, wbuf,
             ae, be, rows, sem_ab0, sem_ab1, sem_r0, sem_r1):
    cid = lax.axis_index("c")
    sid = lax.axis_index("s")
    zero = jnp.zeros((_L,), jnp.float32)

    zero32 = jnp.zeros((32,), jnp.bfloat16)

    def _zb(r, _):
        rows[0, r, :] = zero32
        return _
    lax.fori_loop(0, _CHUNK, _zb, None)
    for k in range(_CHUNK // _L):
        wbuf[0, pl.ds(k * _L, _L)] = zero

    arow = _NP // _NS
    for t in range(arow // _CHUNK):
        pltpu.sync_copy(rows.at[0], accum.at[pl.ds(sid * arow + t * _CHUNK,
                                                   _CHUNK)])
    _atail = arow % _CHUNK
    if _atail:
        pltpu.sync_copy(rows.at[0, pl.ds(0, _atail)],
                        accum.at[pl.ds(sid * arow + arow - _atail, _atail)])

    drow = _NDP // _NS
    for t in range(drow // _CHUNK):
        pltpu.sync_copy(wbuf.at[0],
                        den.at[pl.ds(sid * drow + t * _CHUNK, _CHUNK)])
    _dtail = drow % _CHUNK
    if _dtail:
        pltpu.sync_copy(wbuf.at[0, pl.ds(0, _dtail)],
                        den.at[pl.ds(sid * drow + drow - _dtail, _dtail)])

    @pl.when(sid == 0)
    def _stage():
        pltpu.sync_copy(asrc_h, asrc_s.at[pl.ds(0, _N)])
        pltpu.sync_copy(adst_h, adst_s.at[pl.ds(0, _N)])
        pltpu.sync_copy(wbuf.at[0, pl.ds(0, _NP - _N)],
                        asrc_s.at[pl.ds(_N, _NP - _N)])
        pltpu.sync_copy(wbuf.at[0, pl.ds(0, _NP - _N)],
                        adst_s.at[pl.ds(_N, _NP - _N)])

    plsc.subcore_barrier()

    ebase = sid * _EPT
    dbase = cid * _NHALF
    _SUP = 8
    _SE = _SUP * _CHUNK

    def _compute(jj, p):
        for k in range(_CHUNK // _L):
            s16 = srcw[pl.ds(jj * _CHUNK + k * _L, _L)]
            d16 = dstw[jj, pl.ds(k * _L, _L)]
            x = ae[p, pl.ds(k * _L, _L)] + be[p, pl.ds(k * _L, _L)]
            w = jnp.exp(jnp.where(x >= 0, x, 0.2 * x))
            wbuf[p, pl.ds(k * _L, _L)] = w
            idx2[p, pl.ds(k * _L, _L)] = s16 * 2 + cid
            dl = d16 - dbase
            inr = (dl >= 0) & (dl < _NHALF)
            dstloc[p, pl.ds(k * _L, _L)] = jnp.where(inr, dl, _NHALF)

    def _scale(q):
        def _srow(g, _):
            wv = wbuf[q, pl.ds(g * _L, _L)]
            for t in range(_L):
                r = g * _L + t
                w16 = jnp.broadcast_to(wv[t], (_L,))
                ws = plsc.pack(w16, w16, format=plsc.PackFormat.INTERLEAVED)
                rows[q, r, :] = rows[q, r, :] * ws
            return _
        lax.fori_loop(0, _CHUNK // _L, _srow, None)

    def _issue_ab(jj):
        p = jj % 2
        da = pltpu.async_copy(
            asrc_s.at[srcw.at[pl.ds(jj * _CHUNK, _CHUNK)]],
            ae.at[p], sem_ab0 if p == 0 else sem_ab1)
        db = pltpu.async_copy(
            adst_s.at[dstw.at[jj]],
            be.at[p], sem_ab0 if p == 0 else sem_ab1)
        return da, db

    def _sup(s, _):
        base = ebase + s * _SE
        pltpu.sync_copy(srcp_h.at[pl.ds(base, _SE)], srcw)
        pltpu.sync_copy(dstp2_h.at[pl.ds(sid * _NCHUNK + s * _SUP, _SUP)],
                        dstw)
        descs_ab = [None] * _SUP
        descs_r = [None, None]
        descs_ab[0] = _issue_ab(0)
        for jj in range(_SUP):
            p = jj % 2
            if jj + 1 < _SUP:
                descs_ab[jj + 1] = _issue_ab(jj + 1)
            descs_ab[jj][0].wait()
            descs_ab[jj][1].wait()
            _compute(jj, p)
            descs_r[p] = pltpu.async_copy(
                hv_h.at[idx2.at[p]], rows.at[p],
                sem_r0 if p == 0 else sem_r1)
            if jj > 0:
                q = (jj - 1) % 2
                descs_r[q].wait()
                _scale(q)
                pltpu.sync_copy(rows.at[q], accum.at[dstw.at[jj - 1]],
                                add=True)
                pltpu.sync_copy(wbuf.at[q], den.at[dstloc.at[q]], add=True)
        q = (_SUP - 1) % 2
        descs_r[q].wait()
        _scale(q)
        pltpu.sync_copy(rows.at[q], accum.at[dstw.at[_SUP - 1]], add=True)
        pltpu.sync_copy(wbuf.at[q], den.at[dstloc.at[q]], add=True)
        return _

    lax.fori_loop(0, _EPT // _SE, _sup, None)

    plsc.subcore_barrier()

    pltpu.sync_copy(accum.at[pl.ds(sid * arow, arow)],
                    out_h.at[cid, pl.ds(sid * arow, arow)])
    pltpu.sync_copy(den.at[pl.ds(sid * drow, drow)],
                    den_h.at[cid, pl.ds(sid * drow, drow)])


def _sc_edge(hv, asrc, adst, srcp, dstp2):
    return pl.kernel(
        _sc_body,
        out_type=[jax.ShapeDtypeStruct((_NC, _NP, 32), jnp.bfloat16),
                  jax.ShapeDtypeStruct((_NC, _NDP), jnp.float32)],
        mesh=plsc.VectorSubcoreMesh(core_axis_name="c", subcore_axis_name="s"),
        compiler_params=pltpu.CompilerParams(needs_layout_passes=False,
                                             use_tc_tiling_on_sc=False),
        scratch_types=[
            pltpu.VMEM_SHARED((_NP, 32), jnp.bfloat16),
            pltpu.VMEM_SHARED((_NDP,), jnp.float32),
            pltpu.VMEM_SHARED((_NP,), jnp.float32),
            pltpu.VMEM_SHARED((_NP,), jnp.float32),
            pltpu.VMEM((8 * _CHUNK,), jnp.int32),
            pltpu.VMEM((8, _CHUNK), jnp.int32),
            pltpu.VMEM((2, _CHUNK), jnp.int32),
            pltpu.VMEM((2, _CHUNK), jnp.int32),
            pltpu.VMEM((2, _CHUNK), jnp.float32),
            pltpu.VMEM((2, _CHUNK), jnp.float32),
            pltpu.VMEM((2, _CHUNK), jnp.float32),
            pltpu.VMEM((2, _CHUNK, 32), jnp.bfloat16),
            pltpu.SemaphoreType.DMA,
            pltpu.SemaphoreType.DMA,
            pltpu.SemaphoreType.DMA,
            pltpu.SemaphoreType.DMA,
        ],
    )(hv, asrc, adst, srcp, dstp2)


def _gat_layer(feats, W, a_s, a_d, b, srcp, dstp2):
    d = feats.shape[1]
    c_s = a_s @ W
    c_d = a_d @ W
    waug = (jnp.zeros((d, 128), jnp.float32)
            .at[:, :_H].set(W.T).at[:, _H].set(c_s).at[:, _H + 1].set(c_d))
    out = _linear_aug(feats, waug)
    h = out[:, :_H]
    asrc = out[:, _H]
    adst = out[:, _H + 1]
    hv = h.reshape(2 * _N, 32).astype(jnp.bfloat16)
    agg2, den2 = _sc_edge(hv, asrc, adst, srcp, dstp2)
    agg = jnp.concatenate([agg2[0, :_N], agg2[1, :_N]],
                          axis=1).astype(jnp.float32)
    den = jnp.concatenate([den2[0, :_NHALF], den2[1, :_NHALF]])[:_N]
    return agg / (den + 1e-16)[:, None] + b


def kernel(x, edge_index, batch, W1, a1s, a1d, b1, W2, a2s, a2d, b2, Wl, bl):
    src = edge_index[0]
    dst = edge_index[1]
    srcp = jnp.pad(src, (0, _EP - _E))
    dstp2 = jnp.pad(dst, (0, _EP - _E),
                    constant_values=_TRASH).reshape(_EP // _CHUNK, _CHUNK)
    h = _gat_layer(x, W1, a1s, a1d, b1, srcp, dstp2)
    h = jax.nn.relu(h)
    h = _gat_layer(h, W2, a2s, a2d, b2, srcp, dstp2)
    h = jax.nn.relu(h)
    s = jax.ops.segment_sum(h, batch, num_segments=_G, indices_are_sorted=True)
    cnt = jax.ops.segment_sum(jnp.ones((_N,), jnp.float32), batch,
                              num_segments=_G, indices_are_sorted=True)
    g = s / jnp.maximum(cnt, 1.0)[:, None]
    return g @ Wl.T + bl


# speedup vs baseline: 32.3400x; 1.1125x over previous
import functools

import jax
import jax.numpy as jnp
from jax import lax
from jax.experimental import pallas as pl
from jax.experimental.pallas import tpu as pltpu
from jax.experimental.pallas import tpu_sc as plsc

_N = 50000
_E = 800000
_G = 256
_H = 64
_BLK = 2000

_NC, _NS, _L = 2, 16, 16
_NP = 50048
_TRASH = _N
_EPT = 51200
_EP = _EPT * _NS
_CHUNK = 128
_NCHUNK = _EPT // _CHUNK
_NHALF = _NP // 2
_NDP = _NHALF + 64


def _mm_body(x_ref, w_ref, o_ref):
    o_ref[...] = jnp.dot(x_ref[...], w_ref[...],
                         preferred_element_type=jnp.float32)


def _linear_aug(x, waug):
    n, d = x.shape
    return pl.pallas_call(
        _mm_body,
        grid=(n // _BLK,),
        in_specs=[
            pl.BlockSpec((_BLK, d), lambda i: (i, 0)),
            pl.BlockSpec((d, 128), lambda i: (0, 0)),
        ],
        out_specs=pl.BlockSpec((_BLK, 128), lambda i: (i, 0)),
        out_shape=jax.ShapeDtypeStruct((n, 128), jnp.float32),
    )(x, waug)


def _sc_body(hv_h, asrc_h, adst_h, srcp_h, dstp2_h, out_h, den_h,
             accum, den, asrc_s, adst_s, srcw, dstw, dstloc, idx2, wbuf,
             ae, be, rows, sem_ab0, sem_ab1, sem_r0, sem_r1):
    cid = lax.axis_index("c")
    sid = lax.axis_index("s")
    zero = jnp.zeros((_L,), jnp.float32)

    zero32 = jnp.zeros((32,), jnp.bfloat16)

    def _zb(r, _):
        rows[0, r, :] = zero32
        return _
    lax.fori_loop(0, _CHUNK, _zb, None)
    for k in range(_CHUNK // _L):
        wbuf[0, pl.ds(k * _L, _L)] = zero

    arow = _NP // _NS
    for t in range(arow // _CHUNK):
        pltpu.sync_copy(rows.at[0], accum.at[pl.ds(sid * arow + t * _CHUNK,
                                                   _CHUNK)])
    _atail = arow % _CHUNK
    if _atail:
        pltpu.sync_copy(rows.at[0, pl.ds(0, _atail)],
                        accum.at[pl.ds(sid * arow + arow - _atail, _atail)])

    drow = _NDP // _NS
    for t in range(drow // _CHUNK):
        pltpu.sync_copy(wbuf.at[0],
                        den.at[pl.ds(sid * drow + t * _CHUNK, _CHUNK)])
    _dtail = drow % _CHUNK
    if _dtail:
        pltpu.sync_copy(wbuf.at[0, pl.ds(0, _dtail)],
                        den.at[pl.ds(sid * drow + drow - _dtail, _dtail)])

    @pl.when(sid == 0)
    def _stage():
        pltpu.sync_copy(asrc_h, asrc_s.at[pl.ds(0, _N)])
        pltpu.sync_copy(adst_h, adst_s.at[pl.ds(0, _N)])
        pltpu.sync_copy(wbuf.at[0, pl.ds(0, _NP - _N)],
                        asrc_s.at[pl.ds(_N, _NP - _N)])
        pltpu.sync_copy(wbuf.at[0, pl.ds(0, _NP - _N)],
                        adst_s.at[pl.ds(_N, _NP - _N)])

    plsc.subcore_barrier()

    ebase = sid * _EPT
    dbase = cid * _NHALF
    _SUP = 8
    _SE = _SUP * _CHUNK

    def _compute(jj, p):
        for k in range(_CHUNK // _L):
            s16 = srcw[pl.ds(jj * _CHUNK + k * _L, _L)]
            d16 = dstw[jj, pl.ds(k * _L, _L)]
            x = ae[p, pl.ds(k * _L, _L)] + be[p, pl.ds(k * _L, _L)]
            w = jnp.exp(jnp.where(x >= 0, x, 0.2 * x))
            wbuf[p, pl.ds(k * _L, _L)] = w
            idx2[p, pl.ds(k * _L, _L)] = s16 * 2 + cid
            dl = d16 - dbase
            inr = (dl >= 0) & (dl < _NHALF)
            dstloc[p, pl.ds(k * _L, _L)] = jnp.where(inr, dl, _NHALF)

    def _scale(q):
        def _srow(g, _):
            wv = wbuf[q, pl.ds(g * _L, _L)]
            for t in range(_L):
                r = g * _L + t
                w16 = jnp.broadcast_to(wv[t], (_L,))
                ws = plsc.pack(w16, w16, format=plsc.PackFormat.INTERLEAVED)
                rows[q, r, :] = rows[q, r, :] * ws
            return _
        lax.fori_loop(0, _CHUNK // _L, _srow, None)

    def _issue_ab(jj):
        p = jj % 2
        da = pltpu.async_copy(
            asrc_s.at[srcw.at[pl.ds(jj * _CHUNK, _CHUNK)]],
            ae.at[p], sem_ab0 if p == 0 else sem_ab1)
        db = pltpu.async_copy(
            adst_s.at[dstw.at[jj]],
            be.at[p], sem_ab0 if p == 0 else sem_ab1)
        return da, db

    def _sup(s, _):
        base = ebase + s * _SE
        pltpu.sync_copy(srcp_h.at[pl.ds(base, _SE)], srcw)
        pltpu.sync_copy(dstp2_h.at[pl.ds(sid * _NCHUNK + s * _SUP, _SUP)],
                        dstw)
        descs_ab = [None] * _SUP
        descs_r = [None, None]
        descs_ab[0] = _issue_ab(0)
        for jj in range(_SUP):
            p = jj % 2
            if jj + 1 < _SUP:
                descs_ab[jj + 1] = _issue_ab(jj + 1)
            descs_ab[jj][0].wait()
            descs_ab[jj][1].wait()
            _compute(jj, p)
            descs_r[p] = pltpu.async_copy(
                hv_h.at[idx2.at[p]], rows.at[p],
                sem_r0 if p == 0 else sem_r1)
            if jj > 0:
                q = (jj - 1) % 2
                descs_r[q].wait()
                _scale(q)
                pltpu.sync_copy(rows.at[q], accum.at[dstw.at[jj - 1]],
                                add=True)
                pltpu.sync_copy(wbuf.at[q], den.at[dstloc.at[q]], add=True)
        q = (_SUP - 1) % 2
        descs_r[q].wait()
        _scale(q)
        pltpu.sync_copy(rows.at[q], accum.at[dstw.at[_SUP - 1]], add=True)
        pltpu.sync_copy(wbuf.at[q], den.at[dstloc.at[q]], add=True)
        return _

    lax.fori_loop(0, _EPT // _SE, _sup, None)

    plsc.subcore_barrier()

    pltpu.sync_copy(accum.at[pl.ds(sid * arow, arow)],
                    out_h.at[cid, pl.ds(sid * arow, arow)])
    pltpu.sync_copy(den.at[pl.ds(sid * drow, drow)],
                    den_h.at[cid, pl.ds(sid * drow, drow)])


def _sc_edge(hv, asrc, adst, srcp, dstp2):
    return pl.kernel(
        _sc_body,
        out_type=[jax.ShapeDtypeStruct((_NC, _NP, 32), jnp.bfloat16),
                  jax.ShapeDtypeStruct((_NC, _NDP), jnp.float32)],
        mesh=plsc.VectorSubcoreMesh(core_axis_name="c", subcore_axis_name="s"),
        compiler_params=pltpu.CompilerParams(needs_layout_passes=False,
                                             use_tc_tiling_on_sc=False),
        scratch_types=[
            pltpu.VMEM_SHARED((_NP, 32), jnp.bfloat16),
            pltpu.VMEM_SHARED((_NDP,), jnp.float32),
            pltpu.VMEM_SHARED((_NP,), jnp.float32),
            pltpu.VMEM_SHARED((_NP,), jnp.float32),
            pltpu.VMEM((8 * _CHUNK,), jnp.int32),
            pltpu.VMEM((8, _CHUNK), jnp.int32),
            pltpu.VMEM((2, _CHUNK), jnp.int32),
            pltpu.VMEM((2, _CHUNK), jnp.int32),
            pltpu.VMEM((2, _CHUNK), jnp.float32),
            pltpu.VMEM((2, _CHUNK), jnp.float32),
            pltpu.VMEM((2, _CHUNK), jnp.float32),
            pltpu.VMEM((2, _CHUNK, 32), jnp.bfloat16),
            pltpu.SemaphoreType.DMA,
            pltpu.SemaphoreType.DMA,
            pltpu.SemaphoreType.DMA,
            pltpu.SemaphoreType.DMA,
        ],
    )(hv, asrc, adst, srcp, dstp2)


_PBLK = _NP // 16


def _pool_body(h_ref, b_ref, wl_ref, bl_ref, o_ref, acc, cnt):
    i = pl.program_id(0)

    @pl.when(i == 0)
    def _init():
        acc[...] = jnp.zeros_like(acc)
        cnt[...] = jnp.zeros_like(cnt)

    hb = jnp.maximum(h_ref[...], 0.0)
    b3 = b_ref[0, 0, :]
    iot = lax.broadcasted_iota(jnp.int32, (_G, _PBLK), 0)
    mask = (iot == b3[None, :]).astype(jnp.float32)
    acc[...] += jnp.dot(mask, hb, preferred_element_type=jnp.float32)
    cnt[...] += jnp.broadcast_to(
        jnp.sum(mask, axis=1, keepdims=True), (_G, 128))

    @pl.when(i == pl.num_programs(0) - 1)
    def _fin():
        rec = 1.0 / jnp.maximum(cnt[...], 1.0)
        g = acc[...] * rec[:, :_H]
        o_ref[...] = jnp.dot(g, wl_ref[...],
                             preferred_element_type=jnp.float32) + bl_ref[...]


def _pool_head(hp, batchp, wlp, blp):
    return pl.pallas_call(
        _pool_body,
        grid=(_NP // _PBLK,),
        in_specs=[
            pl.BlockSpec((_PBLK, _H), lambda i: (i, 0)),
            pl.BlockSpec((1, 1, _PBLK), lambda i: (i, 0, 0)),
            pl.BlockSpec((_H, 128), lambda i: (0, 0)),
            pl.BlockSpec((1, 128), lambda i: (0, 0)),
        ],
        out_specs=pl.BlockSpec((_G, 128), lambda i: (0, 0)),
        out_shape=jax.ShapeDtypeStruct((_G, 128), jnp.float32),
        scratch_shapes=[
            pltpu.VMEM((_G, _H), jnp.float32),
            pltpu.VMEM((_G, 128), jnp.float32),
        ],
    )(hp, batchp, wlp, blp)


def _gat_layer(feats, W, a_s, a_d, b, srcp, dstp2):
    d = feats.shape[1]
    c_s = a_s @ W
    c_d = a_d @ W
    waug = (jnp.zeros((d, 128), jnp.float32)
            .at[:, :_H].set(W.T).at[:, _H].set(c_s).at[:, _H + 1].set(c_d))
    out = _linear_aug(feats, waug)
    h = out[:, :_H]
    asrc = out[:, _H]
    adst = out[:, _H + 1]
    hv = h.reshape(2 * _N, 32).astype(jnp.bfloat16)
    agg2, den2 = _sc_edge(hv, asrc, adst, srcp, dstp2)
    agg = jnp.concatenate([agg2[0, :_N], agg2[1, :_N]],
                          axis=1).astype(jnp.float32)
    den = jnp.concatenate([den2[0, :_NHALF], den2[1, :_NHALF]])[:_N]
    return agg / (den + 1e-16)[:, None] + b


def kernel(x, edge_index, batch, W1, a1s, a1d, b1, W2, a2s, a2d, b2, Wl, bl):
    src = edge_index[0]
    dst = edge_index[1]
    srcp = jnp.pad(src, (0, _EP - _E))
    dstp2 = jnp.pad(dst, (0, _EP - _E),
                    constant_values=_TRASH).reshape(_EP // _CHUNK, _CHUNK)
    h = _gat_layer(x, W1, a1s, a1d, b1, srcp, dstp2)
    h = jax.nn.relu(h)
    h = _gat_layer(h, W2, a2s, a2d, b2, srcp, dstp2)
    hp = jnp.pad(h, ((0, _NP - _N), (0, 0)))
    batchp = jnp.pad(batch, (0, _NP - _N),
                     constant_values=_G + 1).reshape(16, 1, _PBLK)
    wlp = jnp.zeros((_H, 128), jnp.float32).at[:, :2].set(Wl.T)
    blp = jnp.zeros((1, 128), jnp.float32).at[0, :2].set(bl)
    out = _pool_head(hp, batchp, wlp, blp)
    return out[:, :2]
